# Initial kernel scaffold; baseline (speedup 1.0000x reference)
import jax
import jax.numpy as jnp
from jax.experimental import pallas as pl


def kernel(tokens, table):
    raise NotImplementedError("write your pallas kernel here")


# speedup vs baseline: 1.1312x; 1.1312x over previous
import functools
import math

import jax
import jax.numpy as jnp
from jax import lax
from jax.experimental import pallas as pl
from jax.experimental.pallas import tpu as pltpu
from jax.experimental.pallas import tpu_sc as plsc

EMB = 64
SCALE = math.sqrt(float(EMB))
NC = 2
NS = 16
NW = NC * NS
LANES = 16


def _xlane_sum(x):
    for d in (1, 2, 4, 8):
        perm = jnp.arange(LANES, dtype=jnp.int32) ^ d
        x = x + x.at[perm].get(mode="promise_in_bounds")
    return x


def _row_update(rows_v, r):
    v0 = rows_v[r, pl.ds(0, 16)]
    v1 = rows_v[r, pl.ds(16, 16)]
    v2 = rows_v[r, pl.ds(32, 16)]
    v3 = rows_v[r, pl.ds(48, 16)]
    ss = v0 * v0 + v1 * v1 + v2 * v2 + v3 * v3
    tv = _xlane_sum(ss)
    m = jnp.maximum(tv, 1.0)
    i = lax.bitcast_convert_type(m, jnp.int32)
    i = jnp.int32(0x5F3759DF) - lax.shift_right_arithmetic(i, 1)
    y = lax.bitcast_convert_type(i, jnp.float32)
    y = y * (1.5 - 0.5 * m * y * y)
    y = y * (1.5 - 0.5 * m * y * y)
    y = y * (1.5 - 0.5 * m * y * y)
    f = jnp.where(tv > 1.0, y, 1.0) * SCALE
    rows_v[r, pl.ds(0, 16)] = v0 * f
    rows_v[r, pl.ds(16, 16)] = v1 * f
    rows_v[r, pl.ds(32, 16)] = v2 * f
    rows_v[r, pl.ds(48, 16)] = v3 * f


@functools.partial(jax.jit, static_argnames=("n", "chunk"))
def _emb_lookup(tokens_flat, table, *, n, chunk):
    per_w = n // NW
    nchunk = per_w // chunk

    mesh = plsc.VectorSubcoreMesh(core_axis_name="c", subcore_axis_name="s")

    @functools.partial(
        pl.kernel,
        mesh=mesh,
        compiler_params=pltpu.CompilerParams(use_tc_tiling_on_sc=False),
        out_type=jax.ShapeDtypeStruct((n, EMB), jnp.float32),
        scratch_types=[
            pltpu.VMEM((chunk,), jnp.int32),
            pltpu.VMEM((chunk, EMB), jnp.float32),
            pltpu.SemaphoreType.DMA,
        ],
    )
    def body(tok_hbm, table_hbm, out_hbm, idx_v, rows_v, sem):
        wid = lax.axis_index("s") * NC + lax.axis_index("c")
        base = wid * per_w

        def do_chunk(g, carry):
            off = base + g * chunk
            pltpu.sync_copy(tok_hbm.at[pl.ds(off, chunk)], idx_v)
            pltpu.async_copy(table_hbm.at[idx_v], rows_v, sem).wait()

            def do_row(r, c2):
                _row_update(rows_v, r)
                return c2

            lax.fori_loop(0, chunk, do_row, 0)
            pltpu.sync_copy(rows_v, out_hbm.at[pl.ds(off, chunk)])
            return carry

        lax.fori_loop(0, nchunk, do_chunk, 0)

    return body(tokens_flat, table)


def kernel(tokens, table):
    b, l = tokens.shape
    flat = tokens.reshape(-1).astype(jnp.int32)
    out = _emb_lookup(flat, table, n=b * l, chunk=512)
    return out.reshape(b, l, EMB)


# speedup vs baseline: 1.6683x; 1.4749x over previous
import functools
import math

import jax
import jax.numpy as jnp
from jax import lax
from jax.experimental import pallas as pl
from jax.experimental.pallas import tpu as pltpu
from jax.experimental.pallas import tpu_sc as plsc

EMB = 64
SCALE = math.sqrt(float(EMB))
NC = 2
NS = 16
NW = NC * NS
LANES = 16


def _xlane_sum(x):
    for d in (1, 2, 4, 8):
        perm = jnp.arange(LANES, dtype=jnp.int32) ^ d
        x = x + x.at[perm].get(mode="promise_in_bounds")
    return x


def _row_update(rows_v, r):
    v0 = rows_v[r, pl.ds(0, 16)]
    v1 = rows_v[r, pl.ds(16, 16)]
    v2 = rows_v[r, pl.ds(32, 16)]
    v3 = rows_v[r, pl.ds(48, 16)]
    ss = v0 * v0 + v1 * v1 + v2 * v2 + v3 * v3
    tv = _xlane_sum(ss)
    m = jnp.maximum(tv, 1.0)
    i = lax.bitcast_convert_type(m, jnp.int32)
    i = jnp.int32(0x5F3759DF) - lax.shift_right_arithmetic(i, 1)
    y = lax.bitcast_convert_type(i, jnp.float32)
    y = y * (1.5 - 0.5 * m * y * y)
    y = y * (1.5 - 0.5 * m * y * y)
    y = y * (1.5 - 0.5 * m * y * y)
    f = jnp.where(tv > 1.0, y, 1.0) * SCALE
    rows_v[r, pl.ds(0, 16)] = v0 * f
    rows_v[r, pl.ds(16, 16)] = v1 * f
    rows_v[r, pl.ds(32, 16)] = v2 * f
    rows_v[r, pl.ds(48, 16)] = v3 * f


@functools.partial(jax.jit, static_argnames=("n", "chunk"))
def _emb_lookup(tokens_flat, table, *, n, chunk):
    per_w = n // NW
    nchunk = per_w // chunk

    mesh = plsc.VectorSubcoreMesh(core_axis_name="c", subcore_axis_name="s")

    @functools.partial(
        pl.kernel,
        mesh=mesh,
        compiler_params=pltpu.CompilerParams(use_tc_tiling_on_sc=False),
        out_type=jax.ShapeDtypeStruct((n, EMB), jnp.float32),
        scratch_types=[
            pltpu.VMEM((chunk,), jnp.int32),
            pltpu.VMEM((chunk, EMB), jnp.float32),
            pltpu.SemaphoreType.DMA,
        ],
    )
    def body(tok_hbm, table_hbm, out_hbm, idx_v, rows_v, sem):
        wid = lax.axis_index("s") * NC + lax.axis_index("c")
        base = wid * per_w

        def do_chunk(g, carry):
            off = base + g * chunk
            pltpu.sync_copy(tok_hbm.at[pl.ds(off, chunk)], idx_v)
            pltpu.async_copy(table_hbm.at[idx_v], rows_v, sem).wait()

            @plsc.parallel_loop(0, chunk, unroll=8)
            def do_row(r):
                _row_update(rows_v, r)
            pltpu.sync_copy(rows_v, out_hbm.at[pl.ds(off, chunk)])
            return carry

        lax.fori_loop(0, nchunk, do_chunk, 0)

    return body(tokens_flat, table)


def kernel(tokens, table):
    b, l = tokens.shape
    flat = tokens.reshape(-1).astype(jnp.int32)
    out = _emb_lookup(flat, table, n=b * l, chunk=512)
    return out.reshape(b, l, EMB)
